# baseline (speedup 1.0000x reference)
import jax
import jax.numpy as jnp
from jax import lax
from jax.experimental import pallas as pl
from jax.experimental.pallas import tpu as pltpu

SUBJ_ID = 0
NEG = -1e30
K = 100
QPAD = 1024
KPAD = 104


def _body(x_ref, verb_ref, boxes_ref, scale_ref,
          scores_ref, labels_ref, vs_ref, boxes_o_ref,
          s2_ref, t_ref):
    Q, C = x_ref.shape[1], x_ref.shape[2]
    x = x_ref[0]
    m = jnp.max(x, axis=1, keepdims=True)
    s = jnp.sum(jnp.exp(x - m), axis=1, keepdims=True)
    lse = m + jnp.log(s)

    s2_ref[:, :] = jnp.full(s2_ref.shape, NEG, jnp.float32)
    s2_ref[0:Q, 0:C] = x - lse

    t_ref[:, :] = jnp.full(t_ref.shape, NEG, jnp.float32)
    t_ref[0:Q, :] = -jnp.log(s)
    eye = (lax.broadcasted_iota(jnp.int32, (128, 128), 0)
           == lax.broadcasted_iota(jnp.int32, (128, 128), 1)).astype(jnp.float32)
    chunks = [
        lax.dot_general(t_ref[i * 128:(i + 1) * 128, :], eye,
                        (((0,), (0,)), ((), ())),
                        preferred_element_type=jnp.float32,
                        precision=lax.Precision.HIGHEST)
        for i in range(QPAD // 128)
    ]
    rm0 = jnp.concatenate(chunks, axis=0)

    slot = (lax.broadcasted_iota(jnp.int32, (8, 128), 0) * 128
            + lax.broadcasted_iota(jnp.int32, (8, 128), 1))
    lane128 = lax.broadcasted_iota(jnp.int32, (1, 128), 1)
    BIG = jnp.int32(2 ** 30)

    def step(k, carry):
        rm, srow, lrow = carry
        gmax = jnp.max(rm)
        qstar = jnp.min(jnp.where(rm == gmax, slot, BIG))
        row = s2_ref[pl.ds(qstar, 1), :]
        nm = jnp.max(row)
        cstar = jnp.min(jnp.where(row == nm, lane128, BIG))
        row2 = jnp.where(lane128 == cstar, NEG, row)
        s2_ref[pl.ds(qstar, 1), :] = row2
        rm = jnp.where(slot == qstar, jnp.max(row2), rm)
        srow = jnp.where(lane128 == k, nm, srow)
        lrow = jnp.where(lane128 == k, cstar, lrow)
        vrow = verb_ref[0, pl.ds(qstar, 1), :]
        prob = jnp.exp(jnp.full(vrow.shape, nm, jnp.float32))
        vs_ref[0, pl.ds(k, 1), :] = prob / (1.0 + jnp.exp(-vrow))
        boxes_o_ref[0, pl.ds(k, 1), :] = boxes_ref[0, pl.ds(qstar, 1), :]
        return rm, srow, lrow

    carry = (rm0,
             jnp.full((1, 128), NEG, jnp.float32),
             jnp.zeros((1, 128), jnp.int32))
    _, srow, lrow = lax.fori_loop(0, K, step, carry)

    scores_ref[0] = jnp.exp(srow)
    labels_ref[0] = lrow

    mi = lax.broadcasted_iota(jnp.int32, (8, 8), 0)
    mj = lax.broadcasted_iota(jnp.int32, (8, 8), 1)
    a = mj - (mi // 4) * 4
    sign = jnp.where(mi % 4 < 2, -0.5, 0.5).astype(jnp.float32)
    M = (jnp.where(a == mi % 2, 1.0, 0.0)
         + jnp.where(a == mi % 2 + 2, sign, 0.0)).astype(jnp.float32)
    raw = boxes_o_ref[0]
    conv = lax.dot_general(raw, M, (((1,), (1,)), ((), ())),
                           preferred_element_type=jnp.float32,
                           precision=lax.Precision.HIGHEST)
    boxes_o_ref[0] = conv * scale_ref[0]


@jax.jit
def kernel(pred_obj_logits, pred_verb_logits, pred_sub_boxes, pred_obj_boxes,
           target_sizes):
    B, Q, C = pred_obj_logits.shape
    V = pred_verb_logits.shape[-1]

    boxes8 = jnp.concatenate([pred_sub_boxes, pred_obj_boxes], axis=-1)
    h = target_sizes[:, 0].astype(jnp.float32)
    w = target_sizes[:, 1].astype(jnp.float32)
    scale8 = jnp.stack([w, h, w, h, w, h, w, h], axis=1)[:, None, :]

    scores_o, labels_o, vs_o, boxes_o = pl.pallas_call(
        _body,
        grid=(B,),
        in_specs=[
            pl.BlockSpec((1, Q, C), lambda b: (b, 0, 0)),
            pl.BlockSpec((1, Q, V), lambda b: (b, 0, 0)),
            pl.BlockSpec((1, Q, 8), lambda b: (b, 0, 0)),
            pl.BlockSpec((1, 1, 8), lambda b: (b, 0, 0)),
        ],
        out_specs=[
            pl.BlockSpec((1, 1, 128), lambda b: (b, 0, 0)),
            pl.BlockSpec((1, 1, 128), lambda b: (b, 0, 0)),
            pl.BlockSpec((1, KPAD, V), lambda b: (b, 0, 0)),
            pl.BlockSpec((1, KPAD, 8), lambda b: (b, 0, 0)),
        ],
        out_shape=[
            jax.ShapeDtypeStruct((B, 1, 128), jnp.float32),
            jax.ShapeDtypeStruct((B, 1, 128), jnp.int32),
            jax.ShapeDtypeStruct((B, KPAD, V), jnp.float32),
            jax.ShapeDtypeStruct((B, KPAD, 8), jnp.float32),
        ],
        scratch_shapes=[
            pltpu.VMEM((904, 128), jnp.float32),
            pltpu.VMEM((QPAD, 1), jnp.float32),
        ],
        compiler_params=pltpu.CompilerParams(
            dimension_semantics=("parallel",)),
    )(pred_obj_logits, pred_verb_logits, boxes8, scale8)

    obj_scores = scores_o[:, 0, :K]
    obj_labels = labels_o[:, 0, :K]
    labels = jnp.concatenate(
        [jnp.full_like(obj_labels, SUBJ_ID), obj_labels], axis=1)
    bx = boxes_o[:, :K, :]
    boxes = jnp.concatenate([bx[:, :, 0:4], bx[:, :, 4:8]], axis=1)
    vs = vs_o[:, :K, :]
    ids = jnp.arange(2 * K)
    return labels, boxes, vs, obj_scores, ids[:K], ids[K:]


# speedup vs baseline: 1.1214x; 1.1214x over previous
import jax
import jax.numpy as jnp
from jax import lax
from jax.experimental import pallas as pl
from jax.experimental.pallas import tpu as pltpu

SUBJ_ID = 0
NEG = -1e30
K = 100
QPAD = 1024
KPAD = 104
IMGS = 4


def _body(x_ref, verb_ref, boxes_ref, scale_ref,
          scores_ref, labels_ref, vs_ref, boxes_o_ref,
          s2_ref, t_ref, qs_ref, ns_ref):
    Q, C = x_ref.shape[1], x_ref.shape[2]
    V = verb_ref.shape[2]
    eye = (lax.broadcasted_iota(jnp.int32, (128, 128), 0)
           == lax.broadcasted_iota(jnp.int32, (128, 128), 1)).astype(jnp.float32)
    lane128 = lax.broadcasted_iota(jnp.int32, (1, 128), 1)
    slot = (lax.broadcasted_iota(jnp.int32, (8, 128), 0) * 128
            + lax.broadcasted_iota(jnp.int32, (8, 128), 1))

    rms = []
    for i in range(IMGS):
        x = x_ref[i]
        m = jnp.max(x, axis=1, keepdims=True)
        s = jnp.sum(jnp.exp(x - m), axis=1, keepdims=True)
        lse = m + jnp.log(s)
        s2_ref[i] = jnp.full((s2_ref.shape[1], s2_ref.shape[2]), NEG,
                             jnp.float32)
        s2_ref[i, 0:Q, 0:C] = x - lse
        t_ref[i] = jnp.full((QPAD, 1), NEG, jnp.float32)
        t_ref[i, 0:Q, :] = -jnp.log(s)
        chunks = [
            lax.dot_general(t_ref[i, c * 128:(c + 1) * 128, :], eye,
                            (((0,), (0,)), ((), ())),
                            preferred_element_type=jnp.float32,
                            precision=lax.Precision.HIGHEST)
            for c in range(QPAD // 128)
        ]
        rms.append(jnp.concatenate(chunks, axis=0))

    BIG = jnp.int32(2 ** 30)

    def step(k, carry):
        out = []
        for i in range(IMGS):
            rm, srow, lrow = carry[3 * i:3 * i + 3]
            gmax = jnp.max(rm)
            qstar = jnp.min(jnp.where(rm == gmax, slot, BIG))
            row = s2_ref[i, pl.ds(qstar, 1), :]
            nm = jnp.max(row)
            cstar = jnp.min(jnp.where(row == nm, lane128, BIG))
            row2 = jnp.where(lane128 == cstar, NEG, row)
            s2_ref[i, pl.ds(qstar, 1), :] = row2
            rm = jnp.where(slot == qstar, jnp.max(row2), rm)
            srow = jnp.where(lane128 == k, nm, srow)
            lrow = jnp.where(lane128 == k, cstar, lrow)
            qs_ref[i, k] = qstar
            ns_ref[i, k] = nm
            out += [rm, srow, lrow]
        return tuple(out)

    carry = ()
    for i in range(IMGS):
        carry += (rms[i],
                  jnp.full((1, 128), NEG, jnp.float32),
                  jnp.zeros((1, 128), jnp.int32))
    carry = lax.fori_loop(0, K, step, carry)
    for i in range(IMGS):
        scores_ref[i, 0, :] = jnp.exp(carry[3 * i + 1][0])
        labels_ref[i, 0, :] = carry[3 * i + 2][0]

    def gather(j, _):
        for i in range(IMGS):
            q = qs_ref[i, j]
            nm = ns_ref[i, j]
            vrow = verb_ref[i, pl.ds(q, 1), :]
            prob = jnp.exp(jnp.full((1, V), nm, jnp.float32))
            vs_ref[i, pl.ds(j, 1), :] = prob / (1.0 + jnp.exp(-vrow))
            boxes_o_ref[i, pl.ds(j, 1), :] = boxes_ref[i, pl.ds(q, 1), :]
        return 0

    lax.fori_loop(0, K, gather, 0)

    mi = lax.broadcasted_iota(jnp.int32, (8, 8), 0)
    mj = lax.broadcasted_iota(jnp.int32, (8, 8), 1)
    a = mj - (mi // 4) * 4
    sign = jnp.where(mi % 4 < 2, -0.5, 0.5).astype(jnp.float32)
    M = (jnp.where(a == mi % 2, 1.0, 0.0)
         + jnp.where(a == mi % 2 + 2, sign, 0.0)).astype(jnp.float32)
    for i in range(IMGS):
        raw = boxes_o_ref[i]
        conv = lax.dot_general(raw, M, (((1,), (1,)), ((), ())),
                               preferred_element_type=jnp.float32,
                               precision=lax.Precision.HIGHEST)
        boxes_o_ref[i] = conv * scale_ref[i]


@jax.jit
def kernel(pred_obj_logits, pred_verb_logits, pred_sub_boxes, pred_obj_boxes,
           target_sizes):
    B, Q, C = pred_obj_logits.shape
    V = pred_verb_logits.shape[-1]

    boxes8 = jnp.concatenate([pred_sub_boxes, pred_obj_boxes], axis=-1)
    h = target_sizes[:, 0].astype(jnp.float32)
    w = target_sizes[:, 1].astype(jnp.float32)
    scale8 = jnp.stack([w, h, w, h, w, h, w, h], axis=1)[:, None, :]

    scores_o, labels_o, vs_o, boxes_o = pl.pallas_call(
        _body,
        grid=(B // IMGS,),
        in_specs=[
            pl.BlockSpec((IMGS, Q, C), lambda b: (b, 0, 0)),
            pl.BlockSpec((IMGS, Q, V), lambda b: (b, 0, 0)),
            pl.BlockSpec((IMGS, Q, 8), lambda b: (b, 0, 0)),
            pl.BlockSpec((IMGS, 1, 8), lambda b: (b, 0, 0)),
        ],
        out_specs=[
            pl.BlockSpec((IMGS, 1, 128), lambda b: (b, 0, 0)),
            pl.BlockSpec((IMGS, 1, 128), lambda b: (b, 0, 0)),
            pl.BlockSpec((IMGS, KPAD, V), lambda b: (b, 0, 0)),
            pl.BlockSpec((IMGS, KPAD, 8), lambda b: (b, 0, 0)),
        ],
        out_shape=[
            jax.ShapeDtypeStruct((B, 1, 128), jnp.float32),
            jax.ShapeDtypeStruct((B, 1, 128), jnp.int32),
            jax.ShapeDtypeStruct((B, KPAD, V), jnp.float32),
            jax.ShapeDtypeStruct((B, KPAD, 8), jnp.float32),
        ],
        scratch_shapes=[
            pltpu.VMEM((IMGS, 904, 128), jnp.float32),
            pltpu.VMEM((IMGS, QPAD, 1), jnp.float32),
            pltpu.SMEM((IMGS, 128), jnp.int32),
            pltpu.SMEM((IMGS, 128), jnp.float32),
        ],
        compiler_params=pltpu.CompilerParams(
            dimension_semantics=("parallel",)),
    )(pred_obj_logits, pred_verb_logits, boxes8, scale8)

    obj_scores = scores_o[:, 0, :K]
    obj_labels = labels_o[:, 0, :K]
    labels = jnp.concatenate(
        [jnp.full_like(obj_labels, SUBJ_ID), obj_labels], axis=1)
    bx = boxes_o[:, :K, :]
    boxes = jnp.concatenate([bx[:, :, 0:4], bx[:, :, 4:8]], axis=1)
    vs = vs_o[:, :K, :]
    ids = jnp.arange(2 * K)
    return labels, boxes, vs, obj_scores, ids[:K], ids[K:]


# speedup vs baseline: 1.1235x; 1.0019x over previous
import jax
import jax.numpy as jnp
from jax import lax
from jax.experimental import pallas as pl
from jax.experimental.pallas import tpu as pltpu

SUBJ_ID = 0
NEG = -1e30
K = 100
QPAD = 1024
KPAD = 104
IMGS = 4


def _body(x_ref, verb_ref, boxes_ref, scale_ref,
          scores_ref, labels_ref, vs_ref, boxes_o_ref,
          *scratch):
    s2_refs = scratch[:IMGS]
    t_ref = scratch[IMGS]
    qs_refs = scratch[IMGS + 1:2 * IMGS + 1]
    ns_refs = scratch[2 * IMGS + 1:3 * IMGS + 1]
    Q, C = x_ref.shape[1], x_ref.shape[2]
    V = verb_ref.shape[2]
    eye = (lax.broadcasted_iota(jnp.int32, (128, 128), 0)
           == lax.broadcasted_iota(jnp.int32, (128, 128), 1)).astype(jnp.float32)
    lane128 = lax.broadcasted_iota(jnp.int32, (1, 128), 1)
    slot = (lax.broadcasted_iota(jnp.int32, (8, 128), 0) * 128
            + lax.broadcasted_iota(jnp.int32, (8, 128), 1))

    rms = []
    for i in range(IMGS):
        x = x_ref[i]
        m = jnp.max(x, axis=1, keepdims=True)
        s = jnp.sum(jnp.exp(x - m), axis=1, keepdims=True)
        lse = m + jnp.log(s)
        s2_refs[i][:, :] = jnp.full(s2_refs[i].shape, NEG, jnp.float32)
        s2_refs[i][0:Q, 0:C] = x - lse
        t_ref[i] = jnp.full((QPAD, 1), NEG, jnp.float32)
        t_ref[i, 0:Q, :] = -jnp.log(s)
        chunks = [
            lax.dot_general(t_ref[i, c * 128:(c + 1) * 128, :], eye,
                            (((0,), (0,)), ((), ())),
                            preferred_element_type=jnp.float32,
                            precision=lax.Precision.HIGHEST)
            for c in range(QPAD // 128)
        ]
        rms.append(jnp.concatenate(chunks, axis=0))

    BIG = jnp.int32(2 ** 30)

    def step(k, carry):
        out = []
        for i in range(IMGS):
            rm, srow, lrow = carry[3 * i:3 * i + 3]
            gmax = jnp.max(rm)
            qstar = jnp.min(jnp.where(rm == gmax, slot, BIG))
            row = s2_refs[i][pl.ds(qstar, 1), :]
            nm = jnp.max(row)
            cstar = jnp.min(jnp.where(row == nm, lane128, BIG))
            row2 = jnp.where(lane128 == cstar, NEG, row)
            s2_refs[i][pl.ds(qstar, 1), :] = row2
            rm = jnp.where(slot == qstar, jnp.max(row2), rm)
            srow = jnp.where(lane128 == k, nm, srow)
            lrow = jnp.where(lane128 == k, cstar, lrow)
            qs_refs[i][k] = qstar
            ns_refs[i][k] = nm
            out += [rm, srow, lrow]
        return tuple(out)

    carry = ()
    for i in range(IMGS):
        carry += (rms[i],
                  jnp.full((1, 128), NEG, jnp.float32),
                  jnp.zeros((1, 128), jnp.int32))
    carry = lax.fori_loop(0, K, step, carry)
    for i in range(IMGS):
        scores_ref[i, 0, :] = jnp.exp(carry[3 * i + 1][0])
        labels_ref[i, 0, :] = carry[3 * i + 2][0]

    def gather(j, _):
        for i in range(IMGS):
            q = qs_refs[i][j]
            nm = ns_refs[i][j]
            vrow = verb_ref[i, pl.ds(q, 1), :]
            prob = jnp.exp(jnp.full((1, V), nm, jnp.float32))
            vs_ref[i, pl.ds(j, 1), :] = prob / (1.0 + jnp.exp(-vrow))
            boxes_o_ref[i, pl.ds(j, 1), :] = boxes_ref[i, pl.ds(q, 1), :]
        return 0

    lax.fori_loop(0, K, gather, 0)

    mi = lax.broadcasted_iota(jnp.int32, (8, 8), 0)
    mj = lax.broadcasted_iota(jnp.int32, (8, 8), 1)
    a = mj - (mi // 4) * 4
    sign = jnp.where(mi % 4 < 2, -0.5, 0.5).astype(jnp.float32)
    M = (jnp.where(a == mi % 2, 1.0, 0.0)
         + jnp.where(a == mi % 2 + 2, sign, 0.0)).astype(jnp.float32)
    for i in range(IMGS):
        raw = boxes_o_ref[i]
        conv = lax.dot_general(raw, M, (((1,), (1,)), ((), ())),
                               preferred_element_type=jnp.float32,
                               precision=lax.Precision.HIGHEST)
        boxes_o_ref[i] = conv * scale_ref[i]


@jax.jit
def kernel(pred_obj_logits, pred_verb_logits, pred_sub_boxes, pred_obj_boxes,
           target_sizes):
    B, Q, C = pred_obj_logits.shape
    V = pred_verb_logits.shape[-1]

    boxes8 = jnp.concatenate([pred_sub_boxes, pred_obj_boxes], axis=-1)
    h = target_sizes[:, 0].astype(jnp.float32)
    w = target_sizes[:, 1].astype(jnp.float32)
    scale8 = jnp.stack([w, h, w, h, w, h, w, h], axis=1)[:, None, :]

    scores_o, labels_o, vs_o, boxes_o = pl.pallas_call(
        _body,
        grid=(B // IMGS,),
        in_specs=[
            pl.BlockSpec((IMGS, Q, C), lambda b: (b, 0, 0)),
            pl.BlockSpec((IMGS, Q, V), lambda b: (b, 0, 0)),
            pl.BlockSpec((IMGS, Q, 8), lambda b: (b, 0, 0)),
            pl.BlockSpec((IMGS, 1, 8), lambda b: (b, 0, 0)),
        ],
        out_specs=[
            pl.BlockSpec((IMGS, 1, 128), lambda b: (b, 0, 0)),
            pl.BlockSpec((IMGS, 1, 128), lambda b: (b, 0, 0)),
            pl.BlockSpec((IMGS, KPAD, V), lambda b: (b, 0, 0)),
            pl.BlockSpec((IMGS, KPAD, 8), lambda b: (b, 0, 0)),
        ],
        out_shape=[
            jax.ShapeDtypeStruct((B, 1, 128), jnp.float32),
            jax.ShapeDtypeStruct((B, 1, 128), jnp.int32),
            jax.ShapeDtypeStruct((B, KPAD, V), jnp.float32),
            jax.ShapeDtypeStruct((B, KPAD, 8), jnp.float32),
        ],
        scratch_shapes=(
            [pltpu.VMEM((904, 128), jnp.float32) for _ in range(IMGS)]
            + [pltpu.VMEM((IMGS, QPAD, 1), jnp.float32)]
            + [pltpu.SMEM((128,), jnp.int32) for _ in range(IMGS)]
            + [pltpu.SMEM((128,), jnp.float32) for _ in range(IMGS)]
        ),
        compiler_params=pltpu.CompilerParams(
            dimension_semantics=("parallel",)),
    )(pred_obj_logits, pred_verb_logits, boxes8, scale8)

    obj_scores = scores_o[:, 0, :K]
    obj_labels = labels_o[:, 0, :K]
    labels = jnp.concatenate(
        [jnp.full_like(obj_labels, SUBJ_ID), obj_labels], axis=1)
    bx = boxes_o[:, :K, :]
    boxes = jnp.concatenate([bx[:, :, 0:4], bx[:, :, 4:8]], axis=1)
    vs = vs_o[:, :K, :]
    ids = jnp.arange(2 * K)
    return labels, boxes, vs, obj_scores, ids[:K], ids[K:]


# speedup vs baseline: 4.0557x; 3.6099x over previous
import functools

import jax
import jax.numpy as jnp
from jax import lax
from jax.experimental import pallas as pl
from jax.experimental.pallas import tpu as pltpu

SUBJ_ID = 0
K = 100
NEGP = -1.0
LO0 = 0.012
HI0 = 1.001
BISECT_ITERS = 30

DOT = functools.partial(lax.dot_general,
                        preferred_element_type=jnp.float32,
                        precision=lax.Precision.HIGHEST)


def _eye():
    return (lax.broadcasted_iota(jnp.int32, (128, 128), 0)
            == lax.broadcasted_iota(jnp.int32, (128, 128), 1)
            ).astype(jnp.float32)


def _t_to_col(a, eye):
    return DOT(eye, a, (((1,), (1,)), ((), ())))


def _t_to_row(a, eye):
    return DOT(a, eye, (((0,), (0,)), ((), ())))


def _bisect(count_ge, lo, hi, target):
    def it(_, c):
        lo, hi = c
        mid = 0.5 * (lo + hi)
        pred = count_ge(mid) >= target
        return jnp.where(pred, mid, lo), jnp.where(pred, hi, mid)
    lo, hi = lax.fori_loop(0, BISECT_ITERS, it, (lo, hi))
    return lo


def _scan_lanes_excl(v, lane):
    incl = v
    for d in (1, 2, 4, 8, 16, 32, 64):
        sh = pltpu.roll(incl, d, 1)
        incl = incl + jnp.where(lane >= d, sh, 0.0)
    return incl - v, incl


def _body(x_ref, verb_ref, boxes_ref, scale_ref,
          scores_ref, labels_ref, vs_ref, boxes_o_ref,
          s2_ref, t_ref):
    Q, C = x_ref.shape[1], x_ref.shape[2]
    V = verb_ref.shape[2]
    eye = _eye()
    lane = lax.broadcasted_iota(jnp.int32, (1, 128), 1)
    lanef = lane.astype(jnp.float32)
    lane16 = lax.broadcasted_iota(jnp.int32, (128, 128), 1)
    col = lax.broadcasted_iota(jnp.int32, (128, 1), 0).astype(jnp.float32)
    ones_col = jnp.full((128, 1), 1.0, jnp.float32)

    x = x_ref[0]
    m = jnp.max(x, axis=1, keepdims=True)
    e = jnp.exp(x - m)
    s = jnp.sum(e, axis=1, keepdims=True)
    s2_ref[:, :] = jnp.full(s2_ref.shape, NEGP, jnp.float32)
    s2_ref[0:Q, 0:C] = e / s
    t_ref[:, :] = jnp.full(t_ref.shape, NEGP, jnp.float32)
    t_ref[0:Q, :] = 1.0 / s

    rm = jnp.concatenate(
        [_t_to_row(t_ref[c * 128:(c + 1) * 128, :], eye) for c in range(8)],
        axis=0)

    lo1 = _bisect(lambda t: jnp.sum((rm >= t).astype(jnp.float32)),
                  jnp.float32(LO0), jnp.float32(HI0), 100.0)
    maskf = (rm >= lo1).astype(jnp.float32)
    lane8 = lax.broadcasted_iota(jnp.int32, (8, 128), 1)
    excl, incl = _scan_lanes_excl(maskf, lane8)
    rowtot = incl[:, 127:128]
    sub = lax.broadcasted_iota(jnp.int32, (8, 1), 0)
    rincl = rowtot
    for d in (1, 2, 4):
        rincl = rincl + jnp.where(sub >= d, pltpu.roll(rincl, d, 0), 0.0)
    ranks = excl + (rincl - rowtot)
    ranksm = jnp.where(maskf > 0.0, ranks, 1e6)
    count1 = jnp.sum(maskf)

    rank_t = _t_to_col(ranksm, eye)
    qlist = jnp.zeros((1, 128), jnp.float32)
    for i in range(8):
        oh = (rank_t[:, i:i + 1] == lanef).astype(jnp.float32)
        qvals = lanef + jnp.float32(128 * i)
        qlist = qlist + DOT(qvals, oh, (((1,), (0,)), ((), ())))
    qlist_t = _t_to_col(qlist, eye)

    scand = jnp.zeros((128, 128), jnp.float32)
    for j in range(8):
        g = (qlist_t == lanef + jnp.float32(128 * j)).astype(jnp.float32)
        scand = scand + DOT(g, s2_ref[j * 128:(j + 1) * 128, :],
                            (((1,), (0,)), ((), ())))
    scand = jnp.where(col < count1, scand, NEGP)

    lo2 = _bisect(lambda t: jnp.sum((scand >= t).astype(jnp.float32)),
                  lo1, jnp.float32(HI0), 100.0)
    emaskf = (scand >= lo2).astype(jnp.float32)
    lexcl, lincl = _scan_lanes_excl(emaskf, lane16)
    rt2 = lincl[:, 127:128]
    cntx = _t_to_row(rt2, eye)
    basex, bincl = _scan_lanes_excl(cntx, lane)
    count2 = jnp.sum(cntx)
    base_col = _t_to_col(basex, eye)
    franks = lexcl + base_col
    franksm = jnp.where(emaskf > 0.0, franks, 1e6)

    oh2 = ((col >= basex) & (col < basex + cntx)).astype(jnp.float32)
    frank_s = DOT(oh2, franksm, (((1,), (0,)), ((), ())))
    vals_s = DOT(oh2, scand, (((1,), (0,)), ((), ())))
    q_s = DOT(oh2, qlist_t, (((1,), (0,)), ((), ())))
    conehot = (frank_s == col).astype(jnp.float32)
    c_s = DOT(conehot, _t_to_col(lanef, eye), (((1,), (0,)), ((), ())))
    v_s = DOT(vals_s * conehot, ones_col, (((1,), (0,)), ((), ())))
    key_s = q_s * 128.0 + c_s

    vrow = _t_to_row(v_s, eye)
    krow = _t_to_row(key_s, eye)
    valid = lanef < count2
    vrow = jnp.where(valid, vrow, NEGP)
    krow = jnp.where(valid, krow, 1e9)

    k = 2
    while k <= 128:
        j = k // 2
        while j >= 1:
            pv = jnp.where((lane & j) == 0, pltpu.roll(vrow, 128 - j, 1),
                           pltpu.roll(vrow, j, 1))
            pk = jnp.where((lane & j) == 0, pltpu.roll(krow, 128 - j, 1),
                           pltpu.roll(krow, j, 1))
            iamlow = (lane & j) == 0
            descb = (lane & k) == 0
            self_better = (vrow > pv) | ((vrow == pv) & (krow < pk))
            keep = iamlow == descb
            vrow = jnp.where(keep == self_better, vrow, pv)
            krow = jnp.where(keep == self_better, krow, pk)
            j //= 2
        k *= 2

    scores_ref[0] = vrow
    ki = krow.astype(jnp.int32)
    qi = ki // 128
    labels_ref[0] = ki - qi * 128

    q_t = _t_to_col(qi.astype(jnp.float32), eye)
    vg = jnp.zeros((128, V), jnp.float32)
    bg = jnp.zeros((128, 8), jnp.float32)
    for j in range(8):
        n = min(128, Q - j * 128)
        if n <= 0:
            break
        ln = lax.broadcasted_iota(jnp.int32, (1, n), 1).astype(jnp.float32)
        g = (q_t == ln + jnp.float32(128 * j)).astype(jnp.float32)
        vg = vg + DOT(g, verb_ref[0, j * 128:j * 128 + n, :],
                      (((1,), (0,)), ((), ())))
        bg = bg + DOT(g, boxes_ref[0, j * 128:j * 128 + n, :],
                      (((1,), (0,)), ((), ())))
    p_col = _t_to_col(vrow, eye)
    vs_ref[0] = p_col / (1.0 + jnp.exp(-vg))

    mi = lax.broadcasted_iota(jnp.int32, (8, 8), 0)
    mj = lax.broadcasted_iota(jnp.int32, (8, 8), 1)
    a = mj - (mi // 4) * 4
    sgn = jnp.where(mi % 4 < 2, -0.5, 0.5).astype(jnp.float32)
    M = (jnp.where(a == mi % 2, 1.0, 0.0)
         + jnp.where(a == mi % 2 + 2, sgn, 0.0)).astype(jnp.float32)
    conv = DOT(bg, M, (((1,), (1,)), ((), ())))
    boxes_o_ref[0] = conv * scale_ref[0]


@jax.jit
def kernel(pred_obj_logits, pred_verb_logits, pred_sub_boxes, pred_obj_boxes,
           target_sizes):
    B, Q, C = pred_obj_logits.shape
    V = pred_verb_logits.shape[-1]

    boxes8 = jnp.concatenate([pred_sub_boxes, pred_obj_boxes], axis=-1)
    h = target_sizes[:, 0].astype(jnp.float32)
    w = target_sizes[:, 1].astype(jnp.float32)
    scale8 = jnp.stack([w, h, w, h, w, h, w, h], axis=1)[:, None, :]

    scores_o, labels_o, vs_o, boxes_o = pl.pallas_call(
        _body,
        grid=(B,),
        in_specs=[
            pl.BlockSpec((1, Q, C), lambda b: (b, 0, 0)),
            pl.BlockSpec((1, Q, V), lambda b: (b, 0, 0)),
            pl.BlockSpec((1, Q, 8), lambda b: (b, 0, 0)),
            pl.BlockSpec((1, 1, 8), lambda b: (b, 0, 0)),
        ],
        out_specs=[
            pl.BlockSpec((1, 1, 128), lambda b: (b, 0, 0)),
            pl.BlockSpec((1, 1, 128), lambda b: (b, 0, 0)),
            pl.BlockSpec((1, 128, V), lambda b: (b, 0, 0)),
            pl.BlockSpec((1, 128, 8), lambda b: (b, 0, 0)),
        ],
        out_shape=[
            jax.ShapeDtypeStruct((B, 1, 128), jnp.float32),
            jax.ShapeDtypeStruct((B, 1, 128), jnp.int32),
            jax.ShapeDtypeStruct((B, 128, V), jnp.float32),
            jax.ShapeDtypeStruct((B, 128, 8), jnp.float32),
        ],
        scratch_shapes=[
            pltpu.VMEM((1024, 128), jnp.float32),
            pltpu.VMEM((1024, 1), jnp.float32),
        ],
        compiler_params=pltpu.CompilerParams(
            dimension_semantics=("parallel",)),
    )(pred_obj_logits, pred_verb_logits, boxes8, scale8)

    obj_scores = scores_o[:, 0, :K]
    obj_labels = labels_o[:, 0, :K]
    labels = jnp.concatenate(
        [jnp.full_like(obj_labels, SUBJ_ID), obj_labels], axis=1)
    bx = boxes_o[:, :K, :]
    boxes = jnp.concatenate([bx[:, :, 0:4], bx[:, :, 4:8]], axis=1)
    vs = vs_o[:, :K, :]
    ids = jnp.arange(2 * K)
    return labels, boxes, vs, obj_scores, ids[:K], ids[K:]


# speedup vs baseline: 4.5608x; 1.1245x over previous
import functools

import jax
import jax.numpy as jnp
from jax import lax
from jax.experimental import pallas as pl
from jax.experimental.pallas import tpu as pltpu

SUBJ_ID = 0
K = 100
NEGP = -1.0
LO0 = 0.012
HI0 = 1.001
BISECT_ITERS = 22

DOT = functools.partial(lax.dot_general,
                        preferred_element_type=jnp.float32,
                        precision=lax.Precision.HIGHEST)


def _eye():
    return (lax.broadcasted_iota(jnp.int32, (128, 128), 0)
            == lax.broadcasted_iota(jnp.int32, (128, 128), 1)
            ).astype(jnp.float32)


def _t_to_col(a, eye):
    return DOT(eye, a, (((1,), (1,)), ((), ())))


def _t_to_row(a, eye):
    return DOT(a, eye, (((0,), (0,)), ((), ())))


def _bisect(arr, lo, hi, target):
    def it(_, c):
        lo, hi = c
        mid = 0.5 * (lo + hi)
        cnt = jnp.sum(jnp.where(arr >= mid, 1.0, 0.0), axis=(0, 1),
                      keepdims=True)
        pred = cnt >= target
        return jnp.where(pred, mid, lo), jnp.where(pred, hi, mid)
    lo, hi = lax.fori_loop(0, BISECT_ITERS, it, (lo, hi))
    return lo


def _scan_lanes_excl(v, lane):
    incl = v
    for d in (1, 2, 4, 8, 16, 32, 64):
        sh = pltpu.roll(incl, d, 1)
        incl = incl + jnp.where(lane >= d, sh, 0.0)
    return incl - v, incl


def _body(x_ref, verb_ref, boxes_ref, scale_ref,
          scores_ref, labels_ref, vs_ref, boxes_o_ref,
          s2_ref, t_ref):
    Q, C = x_ref.shape[1], x_ref.shape[2]
    V = verb_ref.shape[2]
    eye = _eye()
    lane = lax.broadcasted_iota(jnp.int32, (1, 128), 1)
    lanef = lane.astype(jnp.float32)
    lane16 = lax.broadcasted_iota(jnp.int32, (128, 128), 1)
    col = lax.broadcasted_iota(jnp.int32, (128, 1), 0).astype(jnp.float32)
    ones_col = jnp.full((128, 1), 1.0, jnp.float32)

    x = x_ref[0]
    m = jnp.max(x, axis=1, keepdims=True)
    e = jnp.exp(x - m)
    s = jnp.sum(e, axis=1, keepdims=True)
    s2_ref[:, :] = jnp.full(s2_ref.shape, NEGP, jnp.float32)
    s2_ref[0:Q, 0:C] = e / s
    t_ref[:, :] = jnp.full(t_ref.shape, NEGP, jnp.float32)
    t_ref[0:Q, :] = 1.0 / s

    rm = jnp.concatenate(
        [_t_to_row(t_ref[c * 128:(c + 1) * 128, :], eye) for c in range(8)],
        axis=0)

    lo1 = _bisect(rm, jnp.full((1, 1), LO0, jnp.float32),
                  jnp.full((1, 1), HI0, jnp.float32), 100.0)
    maskf = (rm >= lo1).astype(jnp.float32)
    lane8 = lax.broadcasted_iota(jnp.int32, (8, 128), 1)
    excl, incl = _scan_lanes_excl(maskf, lane8)
    rowtot = incl[:, 127:128]
    sub = lax.broadcasted_iota(jnp.int32, (8, 1), 0)
    rincl = rowtot
    for d in (1, 2, 4):
        rincl = rincl + jnp.where(sub >= d, pltpu.roll(rincl, d, 0), 0.0)
    ranks = excl + (rincl - rowtot)
    ranksm = jnp.where(maskf > 0.0, ranks, 1e6)
    count1 = jnp.sum(maskf, axis=(0, 1), keepdims=True)

    rank_t = _t_to_col(ranksm, eye)
    qlist = jnp.zeros((1, 128), jnp.float32)
    for i in range(8):
        oh = (rank_t[:, i:i + 1] == lanef).astype(jnp.float32)
        qvals = lanef + jnp.float32(128 * i)
        qlist = qlist + DOT(qvals, oh, (((1,), (0,)), ((), ())))
    qlist_t = _t_to_col(qlist, eye)

    scand = jnp.zeros((128, 128), jnp.float32)
    for j in range(8):
        g = (qlist_t == lanef + jnp.float32(128 * j)).astype(jnp.float32)
        scand = scand + DOT(g, s2_ref[j * 128:(j + 1) * 128, :],
                            (((1,), (0,)), ((), ())))
    scand = jnp.where(col < count1, scand, NEGP)

    lo2 = _bisect(scand, lo1, jnp.full((1, 1), HI0, jnp.float32), 100.0)
    emaskf = (scand >= lo2).astype(jnp.float32)
    lexcl, lincl = _scan_lanes_excl(emaskf, lane16)
    rt2 = lincl[:, 127:128]
    cntx = _t_to_row(rt2, eye)
    basex, bincl = _scan_lanes_excl(cntx, lane)
    count2 = jnp.sum(cntx, axis=(0, 1), keepdims=True)
    base_col = _t_to_col(basex, eye)
    franks = lexcl + base_col
    franksm = jnp.where(emaskf > 0.0, franks, 1e6)

    oh2 = ((col >= basex) & (col < basex + cntx)).astype(jnp.float32)
    frank_s = DOT(oh2, franksm, (((1,), (0,)), ((), ())))
    vals_s = DOT(oh2, scand, (((1,), (0,)), ((), ())))
    q_s = DOT(oh2, qlist_t, (((1,), (0,)), ((), ())))
    conehot = (frank_s == col).astype(jnp.float32)
    c_s = DOT(conehot, _t_to_col(lanef, eye), (((1,), (0,)), ((), ())))
    v_s = DOT(vals_s * conehot, ones_col, (((1,), (0,)), ((), ())))
    key_s = q_s * 128.0 + c_s

    vrow = _t_to_row(v_s, eye)
    krow = _t_to_row(key_s, eye)
    valid = lanef < count2
    vrow = jnp.where(valid, vrow, NEGP)
    krow = jnp.where(valid, krow, 1e9)

    k = 2
    while k <= 128:
        j = k // 2
        while j >= 1:
            pv = jnp.where((lane & j) == 0, pltpu.roll(vrow, 128 - j, 1),
                           pltpu.roll(vrow, j, 1))
            pk = jnp.where((lane & j) == 0, pltpu.roll(krow, 128 - j, 1),
                           pltpu.roll(krow, j, 1))
            iamlow = (lane & j) == 0
            descb = (lane & k) == 0
            self_better = (vrow > pv) | ((vrow == pv) & (krow < pk))
            keep = iamlow == descb
            vrow = jnp.where(keep == self_better, vrow, pv)
            krow = jnp.where(keep == self_better, krow, pk)
            j //= 2
        k *= 2

    scores_ref[0] = vrow
    ki = krow.astype(jnp.int32)
    qi = ki // 128
    labels_ref[0] = ki - qi * 128

    q_t = _t_to_col(qi.astype(jnp.float32), eye)
    vg = jnp.zeros((128, V), jnp.float32)
    bg = jnp.zeros((128, 8), jnp.float32)
    for j in range(8):
        n = min(128, Q - j * 128)
        if n <= 0:
            break
        ln = lax.broadcasted_iota(jnp.int32, (1, n), 1).astype(jnp.float32)
        g = (q_t == ln + jnp.float32(128 * j)).astype(jnp.float32)
        vg = vg + DOT(g, verb_ref[0, j * 128:j * 128 + n, :],
                      (((1,), (0,)), ((), ())))
        bg = bg + DOT(g, boxes_ref[0, j * 128:j * 128 + n, :],
                      (((1,), (0,)), ((), ())))
    p_col = _t_to_col(vrow, eye)
    vs_ref[0] = p_col / (1.0 + jnp.exp(-vg))

    mi = lax.broadcasted_iota(jnp.int32, (8, 8), 0)
    mj = lax.broadcasted_iota(jnp.int32, (8, 8), 1)
    a = mj - (mi // 4) * 4
    sgn = jnp.where(mi % 4 < 2, -0.5, 0.5).astype(jnp.float32)
    M = (jnp.where(a == mi % 2, 1.0, 0.0)
         + jnp.where(a == mi % 2 + 2, sgn, 0.0)).astype(jnp.float32)
    conv = DOT(bg, M, (((1,), (1,)), ((), ())))
    boxes_o_ref[0] = conv * scale_ref[0]


@jax.jit
def kernel(pred_obj_logits, pred_verb_logits, pred_sub_boxes, pred_obj_boxes,
           target_sizes):
    B, Q, C = pred_obj_logits.shape
    V = pred_verb_logits.shape[-1]

    boxes8 = jnp.concatenate([pred_sub_boxes, pred_obj_boxes], axis=-1)
    h = target_sizes[:, 0].astype(jnp.float32)
    w = target_sizes[:, 1].astype(jnp.float32)
    scale8 = jnp.stack([w, h, w, h, w, h, w, h], axis=1)[:, None, :]

    scores_o, labels_o, vs_o, boxes_o = pl.pallas_call(
        _body,
        grid=(B,),
        in_specs=[
            pl.BlockSpec((1, Q, C), lambda b: (b, 0, 0)),
            pl.BlockSpec((1, Q, V), lambda b: (b, 0, 0)),
            pl.BlockSpec((1, Q, 8), lambda b: (b, 0, 0)),
            pl.BlockSpec((1, 1, 8), lambda b: (b, 0, 0)),
        ],
        out_specs=[
            pl.BlockSpec((1, 1, 128), lambda b: (b, 0, 0)),
            pl.BlockSpec((1, 1, 128), lambda b: (b, 0, 0)),
            pl.BlockSpec((1, 128, V), lambda b: (b, 0, 0)),
            pl.BlockSpec((1, 128, 8), lambda b: (b, 0, 0)),
        ],
        out_shape=[
            jax.ShapeDtypeStruct((B, 1, 128), jnp.float32),
            jax.ShapeDtypeStruct((B, 1, 128), jnp.int32),
            jax.ShapeDtypeStruct((B, 128, V), jnp.float32),
            jax.ShapeDtypeStruct((B, 128, 8), jnp.float32),
        ],
        scratch_shapes=[
            pltpu.VMEM((1024, 128), jnp.float32),
            pltpu.VMEM((1024, 1), jnp.float32),
        ],
        compiler_params=pltpu.CompilerParams(
            dimension_semantics=("parallel",)),
    )(pred_obj_logits, pred_verb_logits, boxes8, scale8)

    obj_scores = scores_o[:, 0, :K]
    obj_labels = labels_o[:, 0, :K]
    labels = jnp.concatenate(
        [jnp.full_like(obj_labels, SUBJ_ID), obj_labels], axis=1)
    bx = boxes_o[:, :K, :]
    boxes = jnp.concatenate([bx[:, :, 0:4], bx[:, :, 4:8]], axis=1)
    vs = vs_o[:, :K, :]
    ids = jnp.arange(2 * K)
    return labels, boxes, vs, obj_scores, ids[:K], ids[K:]


# speedup vs baseline: 5.6217x; 1.2326x over previous
import functools

import jax
import jax.numpy as jnp
from jax import lax
from jax.experimental import pallas as pl
from jax.experimental.pallas import tpu as pltpu

SUBJ_ID = 0
K = 100
NEGP = -1.0
LO0 = 0.012
HI0 = 1.001
BISECT_ITERS = 8

DOT = functools.partial(lax.dot_general,
                        preferred_element_type=jnp.float32,
                        precision=lax.Precision.HIGHEST)


def _eye():
    return (lax.broadcasted_iota(jnp.int32, (128, 128), 0)
            == lax.broadcasted_iota(jnp.int32, (128, 128), 1)
            ).astype(jnp.float32)


def _t_to_col(a, eye):
    return DOT(eye, a, (((1,), (1,)), ((), ())))


def _t_to_row(a, eye):
    return DOT(a, eye, (((0,), (0,)), ((), ())))


def _bisect(arr, lo, hi, target):
    def it(_, c):
        lo, hi = c
        w = (hi - lo) * 0.125
        nlo, nhi = lo, hi
        for m_ in range(1, 8):
            t = lo + w * m_
            cnt = jnp.sum(jnp.where(arr >= t, 1.0, 0.0), axis=(0, 1),
                          keepdims=True)
            pred = cnt >= target
            nlo = jnp.where(pred, t, nlo)
            nhi = jnp.where(pred, nhi, jnp.minimum(nhi, t))
        return nlo, nhi
    lo, hi = lax.fori_loop(0, BISECT_ITERS, it, (lo, hi))
    return lo


def _scan_lanes_excl(v, lane):
    incl = v
    for d in (1, 2, 4, 8, 16, 32, 64):
        sh = pltpu.roll(incl, d, 1)
        incl = incl + jnp.where(lane >= d, sh, 0.0)
    return incl - v, incl


def _body(x_ref, verb_ref, boxes_ref, scale_ref,
          scores_ref, labels_ref, vs_ref, boxes_o_ref,
          s2_ref, t_ref):
    Q, C = x_ref.shape[1], x_ref.shape[2]
    V = verb_ref.shape[2]
    eye = _eye()
    lane = lax.broadcasted_iota(jnp.int32, (1, 128), 1)
    lanef = lane.astype(jnp.float32)
    lane16 = lax.broadcasted_iota(jnp.int32, (128, 128), 1)
    col = lax.broadcasted_iota(jnp.int32, (128, 1), 0).astype(jnp.float32)
    ones_col = jnp.full((128, 1), 1.0, jnp.float32)

    x = x_ref[0]
    m = jnp.max(x, axis=1, keepdims=True)
    e = jnp.exp(x - m)
    s = jnp.sum(e, axis=1, keepdims=True)
    s2_ref[:, :] = jnp.full(s2_ref.shape, NEGP, jnp.float32)
    s2_ref[0:Q, 0:C] = e / s
    t_ref[:, :] = jnp.full(t_ref.shape, NEGP, jnp.float32)
    t_ref[0:Q, :] = 1.0 / s

    rm = jnp.concatenate(
        [_t_to_row(t_ref[c * 128:(c + 1) * 128, :], eye) for c in range(8)],
        axis=0)

    lo1 = _bisect(rm, jnp.full((1, 1), LO0, jnp.float32),
                  jnp.full((1, 1), HI0, jnp.float32), 100.0)
    maskf = (rm >= lo1).astype(jnp.float32)
    lane8 = lax.broadcasted_iota(jnp.int32, (8, 128), 1)
    excl, incl = _scan_lanes_excl(maskf, lane8)
    rowtot = incl[:, 127:128]
    sub = lax.broadcasted_iota(jnp.int32, (8, 1), 0)
    rincl = rowtot
    for d in (1, 2, 4):
        rincl = rincl + jnp.where(sub >= d, pltpu.roll(rincl, d, 0), 0.0)
    ranks = excl + (rincl - rowtot)
    ranksm = jnp.where(maskf > 0.0, ranks, 1e6)
    count1 = jnp.sum(maskf, axis=(0, 1), keepdims=True)

    rank_t = _t_to_col(ranksm, eye)
    qlist = jnp.zeros((1, 128), jnp.float32)
    for i in range(8):
        oh = (rank_t[:, i:i + 1] == lanef).astype(jnp.float32)
        qvals = lanef + jnp.float32(128 * i)
        qlist = qlist + DOT(qvals, oh, (((1,), (0,)), ((), ())))
    qlist_t = _t_to_col(qlist, eye)

    scand = jnp.zeros((128, 128), jnp.float32)
    for j in range(8):
        g = (qlist_t == lanef + jnp.float32(128 * j)).astype(jnp.float32)
        scand = scand + DOT(g, s2_ref[j * 128:(j + 1) * 128, :],
                            (((1,), (0,)), ((), ())))
    scand = jnp.where(col < count1, scand, NEGP)

    lo2 = _bisect(scand, lo1, jnp.full((1, 1), HI0, jnp.float32), 100.0)
    emaskf = (scand >= lo2).astype(jnp.float32)
    lexcl, lincl = _scan_lanes_excl(emaskf, lane16)
    rt2 = lincl[:, 127:128]
    cntx = _t_to_row(rt2, eye)
    basex, bincl = _scan_lanes_excl(cntx, lane)
    count2 = jnp.sum(cntx, axis=(0, 1), keepdims=True)
    base_col = _t_to_col(basex, eye)
    franks = lexcl + base_col
    franksm = jnp.where(emaskf > 0.0, franks, 1e6)

    oh2 = ((col >= basex) & (col < basex + cntx)).astype(jnp.float32)
    frank_s = DOT(oh2, franksm, (((1,), (0,)), ((), ())))
    vals_s = DOT(oh2, scand, (((1,), (0,)), ((), ())))
    q_s = DOT(oh2, qlist_t, (((1,), (0,)), ((), ())))
    conehot = (frank_s == col).astype(jnp.float32)
    c_s = DOT(conehot, _t_to_col(lanef, eye), (((1,), (0,)), ((), ())))
    v_s = DOT(vals_s * conehot, ones_col, (((1,), (0,)), ((), ())))
    key_s = q_s * 128.0 + c_s

    vrow = _t_to_row(v_s, eye)
    krow = _t_to_row(key_s, eye)
    valid = lanef < count2
    vrow = jnp.where(valid, vrow, NEGP)
    krow = jnp.where(valid, krow, 1e9)

    k = 2
    while k <= 128:
        j = k // 2
        while j >= 1:
            pv = jnp.where((lane & j) == 0, pltpu.roll(vrow, 128 - j, 1),
                           pltpu.roll(vrow, j, 1))
            pk = jnp.where((lane & j) == 0, pltpu.roll(krow, 128 - j, 1),
                           pltpu.roll(krow, j, 1))
            iamlow = (lane & j) == 0
            descb = (lane & k) == 0
            self_better = (vrow > pv) | ((vrow == pv) & (krow < pk))
            keep = iamlow == descb
            vrow = jnp.where(keep == self_better, vrow, pv)
            krow = jnp.where(keep == self_better, krow, pk)
            j //= 2
        k *= 2

    scores_ref[0] = vrow
    ki = krow.astype(jnp.int32)
    qi = ki // 128
    labels_ref[0] = ki - qi * 128

    q_t = _t_to_col(qi.astype(jnp.float32), eye)
    vg = jnp.zeros((128, V), jnp.float32)
    bg = jnp.zeros((128, 8), jnp.float32)
    for j in range(8):
        n = min(128, Q - j * 128)
        if n <= 0:
            break
        ln = lax.broadcasted_iota(jnp.int32, (1, n), 1).astype(jnp.float32)
        g = (q_t == ln + jnp.float32(128 * j)).astype(jnp.float32)
        vg = vg + DOT(g, verb_ref[0, j * 128:j * 128 + n, :],
                      (((1,), (0,)), ((), ())))
        bg = bg + DOT(g, boxes_ref[0, j * 128:j * 128 + n, :],
                      (((1,), (0,)), ((), ())))
    p_col = _t_to_col(vrow, eye)
    vs_ref[0] = p_col / (1.0 + jnp.exp(-vg))

    mi = lax.broadcasted_iota(jnp.int32, (8, 8), 0)
    mj = lax.broadcasted_iota(jnp.int32, (8, 8), 1)
    a = mj - (mi // 4) * 4
    sgn = jnp.where(mi % 4 < 2, -0.5, 0.5).astype(jnp.float32)
    M = (jnp.where(a == mi % 2, 1.0, 0.0)
         + jnp.where(a == mi % 2 + 2, sgn, 0.0)).astype(jnp.float32)
    conv = DOT(bg, M, (((1,), (1,)), ((), ())))
    boxes_o_ref[0] = conv * scale_ref[0]


@jax.jit
def kernel(pred_obj_logits, pred_verb_logits, pred_sub_boxes, pred_obj_boxes,
           target_sizes):
    B, Q, C = pred_obj_logits.shape
    V = pred_verb_logits.shape[-1]

    boxes8 = jnp.concatenate([pred_sub_boxes, pred_obj_boxes], axis=-1)
    h = target_sizes[:, 0].astype(jnp.float32)
    w = target_sizes[:, 1].astype(jnp.float32)
    scale8 = jnp.stack([w, h, w, h, w, h, w, h], axis=1)[:, None, :]

    scores_o, labels_o, vs_o, boxes_o = pl.pallas_call(
        _body,
        grid=(B,),
        in_specs=[
            pl.BlockSpec((1, Q, C), lambda b: (b, 0, 0)),
            pl.BlockSpec((1, Q, V), lambda b: (b, 0, 0)),
            pl.BlockSpec((1, Q, 8), lambda b: (b, 0, 0)),
            pl.BlockSpec((1, 1, 8), lambda b: (b, 0, 0)),
        ],
        out_specs=[
            pl.BlockSpec((1, 1, 128), lambda b: (b, 0, 0)),
            pl.BlockSpec((1, 1, 128), lambda b: (b, 0, 0)),
            pl.BlockSpec((1, 128, V), lambda b: (b, 0, 0)),
            pl.BlockSpec((1, 128, 8), lambda b: (b, 0, 0)),
        ],
        out_shape=[
            jax.ShapeDtypeStruct((B, 1, 128), jnp.float32),
            jax.ShapeDtypeStruct((B, 1, 128), jnp.int32),
            jax.ShapeDtypeStruct((B, 128, V), jnp.float32),
            jax.ShapeDtypeStruct((B, 128, 8), jnp.float32),
        ],
        scratch_shapes=[
            pltpu.VMEM((1024, 128), jnp.float32),
            pltpu.VMEM((1024, 1), jnp.float32),
        ],
        compiler_params=pltpu.CompilerParams(
            dimension_semantics=("parallel",)),
    )(pred_obj_logits, pred_verb_logits, boxes8, scale8)

    obj_scores = scores_o[:, 0, :K]
    obj_labels = labels_o[:, 0, :K]
    labels = jnp.concatenate(
        [jnp.full_like(obj_labels, SUBJ_ID), obj_labels], axis=1)
    bx = boxes_o[:, :K, :]
    boxes = jnp.concatenate([bx[:, :, 0:4], bx[:, :, 4:8]], axis=1)
    vs = vs_o[:, :K, :]
    ids = jnp.arange(2 * K)
    return labels, boxes, vs, obj_scores, ids[:K], ids[K:]


# speedup vs baseline: 6.6554x; 1.1839x over previous
import functools

import jax
import jax.numpy as jnp
from jax import lax
from jax.experimental import pallas as pl
from jax.experimental.pallas import tpu as pltpu

SUBJ_ID = 0
K = 100
NEGP = -1.0
LO0 = 0.012
HI0 = 1.001
BISECT_ITERS = 8
IMGS = 2

DOT = functools.partial(lax.dot_general,
                        preferred_element_type=jnp.float32,
                        precision=lax.Precision.HIGHEST)


def _eye():
    return (lax.broadcasted_iota(jnp.int32, (128, 128), 0)
            == lax.broadcasted_iota(jnp.int32, (128, 128), 1)
            ).astype(jnp.float32)


def _t_to_col(a, eye):
    return DOT(eye, a, (((1,), (1,)), ((), ())))


def _t_to_row(a, eye):
    return DOT(a, eye, (((0,), (0,)), ((), ())))


def _bisect_multi(arrs, los, his, target):
    n = len(arrs)

    def it(_, c):
        out = []
        for i in range(n):
            lo, hi = c[2 * i], c[2 * i + 1]
            w = (hi - lo) * 0.125
            nlo, nhi = lo, hi
            for m_ in range(1, 8):
                t = lo + w * m_
                cnt = jnp.sum(jnp.where(arrs[i] >= t, 1.0, 0.0),
                              axis=(0, 1), keepdims=True)
                pred = cnt >= target
                nlo = jnp.where(pred, t, nlo)
                nhi = jnp.where(pred, nhi, jnp.minimum(nhi, t))
            out += [nlo, nhi]
        return tuple(out)

    c0 = ()
    for i in range(n):
        c0 += (los[i], his[i])
    c = lax.fori_loop(0, BISECT_ITERS, it, c0)
    return [c[2 * i] for i in range(n)]


def _scan_lanes_excl(v, lane):
    incl = v
    for d in (1, 2, 4, 8, 16, 32, 64):
        sh = pltpu.roll(incl, d, 1)
        incl = incl + jnp.where(lane >= d, sh, 0.0)
    return incl - v, incl


def _body(x_ref, verb_ref, boxes_ref, scale_ref,
          scores_ref, labels_ref, vs_ref, boxes_o_ref,
          *scratch):
    s2_refs = scratch[:IMGS]
    t_refs = scratch[IMGS:]
    Q, C = x_ref.shape[1], x_ref.shape[2]
    V = verb_ref.shape[2]
    eye = _eye()
    lane = lax.broadcasted_iota(jnp.int32, (1, 128), 1)
    lanef = lane.astype(jnp.float32)
    lane16 = lax.broadcasted_iota(jnp.int32, (128, 128), 1)
    col = lax.broadcasted_iota(jnp.int32, (128, 1), 0).astype(jnp.float32)
    ones_col = jnp.full((128, 1), 1.0, jnp.float32)
    lane8 = lax.broadcasted_iota(jnp.int32, (8, 128), 1)
    sub = lax.broadcasted_iota(jnp.int32, (8, 1), 0)

    rms = []
    for i in range(IMGS):
        x = x_ref[i]
        m = jnp.max(x, axis=1, keepdims=True)
        e = jnp.exp(x - m)
        s = jnp.sum(e, axis=1, keepdims=True)
        s2_refs[i][:, :] = jnp.full(s2_refs[i].shape, NEGP, jnp.float32)
        s2_refs[i][0:Q, 0:C] = e / s
        t_refs[i][:, :] = jnp.full(t_refs[i].shape, NEGP, jnp.float32)
        t_refs[i][0:Q, :] = 1.0 / s
        rms.append(jnp.concatenate(
            [_t_to_row(t_refs[i][c * 128:(c + 1) * 128, :], eye)
             for c in range(8)], axis=0))

    lo1s = _bisect_multi(
        rms,
        [jnp.full((1, 1), LO0, jnp.float32)] * IMGS,
        [jnp.full((1, 1), HI0, jnp.float32)] * IMGS, 100.0)

    scands, qlist_ts, count1s = [], [], []
    for i in range(IMGS):
        maskf = (rms[i] >= lo1s[i]).astype(jnp.float32)
        excl, incl = _scan_lanes_excl(maskf, lane8)
        rowtot = incl[:, 127:128]
        rincl = rowtot
        for d in (1, 2, 4):
            rincl = rincl + jnp.where(sub >= d, pltpu.roll(rincl, d, 0), 0.0)
        ranks = excl + (rincl - rowtot)
        ranksm = jnp.where(maskf > 0.0, ranks, 1e6)
        count1 = jnp.sum(maskf, axis=(0, 1), keepdims=True)

        rank_t = _t_to_col(ranksm, eye)
        qlist = jnp.zeros((1, 128), jnp.float32)
        for c in range(8):
            oh = (rank_t[:, c:c + 1] == lanef).astype(jnp.float32)
            qvals = lanef + jnp.float32(128 * c)
            qlist = qlist + DOT(qvals, oh, (((1,), (0,)), ((), ())))
        qlist_t = _t_to_col(qlist, eye)

        scand = jnp.zeros((128, 128), jnp.float32)
        for j in range(8):
            g = (qlist_t == lanef + jnp.float32(128 * j)).astype(jnp.float32)
            scand = scand + DOT(g, s2_refs[i][j * 128:(j + 1) * 128, :],
                                (((1,), (0,)), ((), ())))
        scand = jnp.where(col < count1, scand, NEGP)
        scands.append(scand)
        qlist_ts.append(qlist_t)
        count1s.append(count1)

    lo2s = _bisect_multi(scands, lo1s,
                         [jnp.full((1, 1), HI0, jnp.float32)] * IMGS, 100.0)

    for i in range(IMGS):
        scand, qlist_t = scands[i], qlist_ts[i]
        emaskf = (scand >= lo2s[i]).astype(jnp.float32)
        lexcl, lincl = _scan_lanes_excl(emaskf, lane16)
        rt2 = lincl[:, 127:128]
        cntx = _t_to_row(rt2, eye)
        basex, _ = _scan_lanes_excl(cntx, lane)
        count2 = jnp.sum(cntx, axis=(0, 1), keepdims=True)
        base_col = _t_to_col(basex, eye)
        franks = lexcl + base_col
        franksm = jnp.where(emaskf > 0.0, franks, 1e6)

        oh2 = ((col >= basex) & (col < basex + cntx)).astype(jnp.float32)
        frank_s = DOT(oh2, franksm, (((1,), (0,)), ((), ())))
        vals_s = DOT(oh2, scand, (((1,), (0,)), ((), ())))
        q_s = DOT(oh2, qlist_t, (((1,), (0,)), ((), ())))
        conehot = (frank_s == col).astype(jnp.float32)
        c_s = DOT(conehot, _t_to_col(lanef, eye), (((1,), (0,)), ((), ())))
        v_s = DOT(vals_s * conehot, ones_col, (((1,), (0,)), ((), ())))
        key_s = q_s * 128.0 + c_s

        vrow = _t_to_row(v_s, eye)
        krow = _t_to_row(key_s, eye)
        valid = lanef < count2
        vrow = jnp.where(valid, vrow, NEGP)
        krow = jnp.where(valid, krow, 1e9)

        k = 2
        while k <= 128:
            j = k // 2
            while j >= 1:
                pv = jnp.where((lane & j) == 0,
                               pltpu.roll(vrow, 128 - j, 1),
                               pltpu.roll(vrow, j, 1))
                pk = jnp.where((lane & j) == 0,
                               pltpu.roll(krow, 128 - j, 1),
                               pltpu.roll(krow, j, 1))
                iamlow = (lane & j) == 0
                descb = (lane & k) == 0
                self_better = (vrow > pv) | ((vrow == pv) & (krow < pk))
                keep = iamlow == descb
                vrow = jnp.where(keep == self_better, vrow, pv)
                krow = jnp.where(keep == self_better, krow, pk)
                j //= 2
            k *= 2

        scores_ref[i] = vrow
        ki = krow.astype(jnp.int32)
        qi = ki // 128
        labels_ref[i] = ki - qi * 128

        q_t = _t_to_col(qi.astype(jnp.float32), eye)
        vg = jnp.zeros((128, V), jnp.float32)
        bg = jnp.zeros((128, 8), jnp.float32)
        for j in range(8):
            n = min(128, Q - j * 128)
            if n <= 0:
                break
            ln = lax.broadcasted_iota(jnp.int32, (1, n), 1).astype(jnp.float32)
            g = (q_t == ln + jnp.float32(128 * j)).astype(jnp.float32)
            vg = vg + DOT(g, verb_ref[i, j * 128:j * 128 + n, :],
                          (((1,), (0,)), ((), ())))
            bg = bg + DOT(g, boxes_ref[i, j * 128:j * 128 + n, :],
                          (((1,), (0,)), ((), ())))
        p_col = _t_to_col(vrow, eye)
        vs_ref[i] = p_col / (1.0 + jnp.exp(-vg))

        mi = lax.broadcasted_iota(jnp.int32, (8, 8), 0)
        mj = lax.broadcasted_iota(jnp.int32, (8, 8), 1)
        a = mj - (mi // 4) * 4
        sgn = jnp.where(mi % 4 < 2, -0.5, 0.5).astype(jnp.float32)
        M = (jnp.where(a == mi % 2, 1.0, 0.0)
             + jnp.where(a == mi % 2 + 2, sgn, 0.0)).astype(jnp.float32)
        conv = DOT(bg, M, (((1,), (1,)), ((), ())))
        boxes_o_ref[i] = conv * scale_ref[i]


@jax.jit
def kernel(pred_obj_logits, pred_verb_logits, pred_sub_boxes, pred_obj_boxes,
           target_sizes):
    B, Q, C = pred_obj_logits.shape
    V = pred_verb_logits.shape[-1]

    boxes8 = jnp.concatenate([pred_sub_boxes, pred_obj_boxes], axis=-1)
    h = target_sizes[:, 0].astype(jnp.float32)
    w = target_sizes[:, 1].astype(jnp.float32)
    scale8 = jnp.stack([w, h, w, h, w, h, w, h], axis=1)[:, None, :]

    scores_o, labels_o, vs_o, boxes_o = pl.pallas_call(
        _body,
        grid=(B // IMGS,),
        in_specs=[
            pl.BlockSpec((IMGS, Q, C), lambda b: (b, 0, 0)),
            pl.BlockSpec((IMGS, Q, V), lambda b: (b, 0, 0)),
            pl.BlockSpec((IMGS, Q, 8), lambda b: (b, 0, 0)),
            pl.BlockSpec((IMGS, 1, 8), lambda b: (b, 0, 0)),
        ],
        out_specs=[
            pl.BlockSpec((IMGS, 1, 128), lambda b: (b, 0, 0)),
            pl.BlockSpec((IMGS, 1, 128), lambda b: (b, 0, 0)),
            pl.BlockSpec((IMGS, 128, V), lambda b: (b, 0, 0)),
            pl.BlockSpec((IMGS, 128, 8), lambda b: (b, 0, 0)),
        ],
        out_shape=[
            jax.ShapeDtypeStruct((B, 1, 128), jnp.float32),
            jax.ShapeDtypeStruct((B, 1, 128), jnp.int32),
            jax.ShapeDtypeStruct((B, 128, V), jnp.float32),
            jax.ShapeDtypeStruct((B, 128, 8), jnp.float32),
        ],
        scratch_shapes=(
            [pltpu.VMEM((1024, 128), jnp.float32) for _ in range(IMGS)]
            + [pltpu.VMEM((1024, 1), jnp.float32) for _ in range(IMGS)]
        ),
        compiler_params=pltpu.CompilerParams(
            dimension_semantics=("parallel",)),
    )(pred_obj_logits, pred_verb_logits, boxes8, scale8)

    obj_scores = scores_o[:, 0, :K]
    obj_labels = labels_o[:, 0, :K]
    labels = jnp.concatenate(
        [jnp.full_like(obj_labels, SUBJ_ID), obj_labels], axis=1)
    bx = boxes_o[:, :K, :]
    boxes = jnp.concatenate([bx[:, :, 0:4], bx[:, :, 4:8]], axis=1)
    vs = vs_o[:, :K, :]
    ids = jnp.arange(2 * K)
    return labels, boxes, vs, obj_scores, ids[:K], ids[K:]


# speedup vs baseline: 7.3175x; 1.0995x over previous
import functools

import jax
import jax.numpy as jnp
from jax import lax
from jax.experimental import pallas as pl
from jax.experimental.pallas import tpu as pltpu

SUBJ_ID = 0
K = 100
NEGP = -1.0
LO0 = 0.012
HI0 = 1.001
BISECT_ITERS = 8
IMGS = 4

DOT = functools.partial(lax.dot_general,
                        preferred_element_type=jnp.float32,
                        precision=lax.Precision.HIGHEST)


def _eye():
    return (lax.broadcasted_iota(jnp.int32, (128, 128), 0)
            == lax.broadcasted_iota(jnp.int32, (128, 128), 1)
            ).astype(jnp.float32)


def _t_to_col(a, eye):
    return DOT(eye, a, (((1,), (1,)), ((), ())))


def _t_to_row(a, eye):
    return DOT(a, eye, (((0,), (0,)), ((), ())))


def _bisect_multi(arrs, los, his, target):
    n = len(arrs)

    def it(_, c):
        out = []
        for i in range(n):
            lo, hi = c[2 * i], c[2 * i + 1]
            w = (hi - lo) * 0.125
            nlo, nhi = lo, hi
            for m_ in range(1, 8):
                t = lo + w * m_
                cnt = jnp.sum(jnp.where(arrs[i] >= t, 1.0, 0.0),
                              axis=(0, 1), keepdims=True)
                pred = cnt >= target
                nlo = jnp.where(pred, t, nlo)
                nhi = jnp.where(pred, nhi, jnp.minimum(nhi, t))
            out += [nlo, nhi]
        return tuple(out)

    c0 = ()
    for i in range(n):
        c0 += (los[i], his[i])
    c = lax.fori_loop(0, BISECT_ITERS, it, c0)
    return [c[2 * i] for i in range(n)]


def _scan_lanes_excl(v, lane):
    incl = v
    for d in (1, 2, 4, 8, 16, 32, 64):
        sh = pltpu.roll(incl, d, 1)
        incl = incl + jnp.where(lane >= d, sh, 0.0)
    return incl - v, incl


def _body(x_ref, verb_ref, boxes_ref, scale_ref,
          scores_ref, labels_ref, vs_ref, boxes_o_ref,
          *scratch):
    s2_refs = scratch[:IMGS]
    t_refs = scratch[IMGS:]
    Q, C = x_ref.shape[1], x_ref.shape[2]
    V = verb_ref.shape[2]
    eye = _eye()
    lane = lax.broadcasted_iota(jnp.int32, (1, 128), 1)
    lanef = lane.astype(jnp.float32)
    lane16 = lax.broadcasted_iota(jnp.int32, (128, 128), 1)
    col = lax.broadcasted_iota(jnp.int32, (128, 1), 0).astype(jnp.float32)
    ones_col = jnp.full((128, 1), 1.0, jnp.float32)
    lane8 = lax.broadcasted_iota(jnp.int32, (8, 128), 1)
    sub = lax.broadcasted_iota(jnp.int32, (8, 1), 0)

    rms = []
    for i in range(IMGS):
        x = x_ref[i]
        m = jnp.max(x, axis=1, keepdims=True)
        e = jnp.exp(x - m)
        s = jnp.sum(e, axis=1, keepdims=True)
        s2_refs[i][:, :] = jnp.full(s2_refs[i].shape, NEGP, jnp.float32)
        s2_refs[i][0:Q, 0:C] = e / s
        t_refs[i][:, :] = jnp.full(t_refs[i].shape, NEGP, jnp.float32)
        t_refs[i][0:Q, :] = 1.0 / s
        rms.append(jnp.concatenate(
            [_t_to_row(t_refs[i][c * 128:(c + 1) * 128, :], eye)
             for c in range(8)], axis=0))

    lo1s = _bisect_multi(
        rms,
        [jnp.full((1, 1), LO0, jnp.float32)] * IMGS,
        [jnp.full((1, 1), HI0, jnp.float32)] * IMGS, 100.0)

    scands, qlist_ts, count1s = [], [], []
    for i in range(IMGS):
        maskf = (rms[i] >= lo1s[i]).astype(jnp.float32)
        excl, incl = _scan_lanes_excl(maskf, lane8)
        rowtot = incl[:, 127:128]
        rincl = rowtot
        for d in (1, 2, 4):
            rincl = rincl + jnp.where(sub >= d, pltpu.roll(rincl, d, 0), 0.0)
        ranks = excl + (rincl - rowtot)
        ranksm = jnp.where(maskf > 0.0, ranks, 1e6)
        count1 = jnp.sum(maskf, axis=(0, 1), keepdims=True)

        rank_t = _t_to_col(ranksm, eye)
        qlist = jnp.zeros((1, 128), jnp.float32)
        for c in range(8):
            oh = (rank_t[:, c:c + 1] == lanef).astype(jnp.float32)
            qvals = lanef + jnp.float32(128 * c)
            qlist = qlist + DOT(qvals, oh, (((1,), (0,)), ((), ())))
        qlist_t = _t_to_col(qlist, eye)

        scand = jnp.zeros((128, 128), jnp.float32)
        for j in range(8):
            g = (qlist_t == lanef + jnp.float32(128 * j)).astype(jnp.float32)
            scand = scand + DOT(g, s2_refs[i][j * 128:(j + 1) * 128, :],
                                (((1,), (0,)), ((), ())))
        scand = jnp.where(col < count1, scand, NEGP)
        scands.append(scand)
        qlist_ts.append(qlist_t)
        count1s.append(count1)

    lo2s = _bisect_multi(scands, lo1s,
                         [jnp.full((1, 1), HI0, jnp.float32)] * IMGS, 100.0)

    for i in range(IMGS):
        scand, qlist_t = scands[i], qlist_ts[i]
        emaskf = (scand >= lo2s[i]).astype(jnp.float32)
        lexcl, lincl = _scan_lanes_excl(emaskf, lane16)
        rt2 = lincl[:, 127:128]
        cntx = _t_to_row(rt2, eye)
        basex, _ = _scan_lanes_excl(cntx, lane)
        count2 = jnp.sum(cntx, axis=(0, 1), keepdims=True)
        base_col = _t_to_col(basex, eye)
        franks = lexcl + base_col
        franksm = jnp.where(emaskf > 0.0, franks, 1e6)

        oh2 = ((col >= basex) & (col < basex + cntx)).astype(jnp.float32)
        frank_s = DOT(oh2, franksm, (((1,), (0,)), ((), ())))
        vals_s = DOT(oh2, scand, (((1,), (0,)), ((), ())))
        q_s = DOT(oh2, qlist_t, (((1,), (0,)), ((), ())))
        conehot = (frank_s == col).astype(jnp.float32)
        c_s = DOT(conehot, _t_to_col(lanef, eye), (((1,), (0,)), ((), ())))
        v_s = DOT(vals_s * conehot, ones_col, (((1,), (0,)), ((), ())))
        key_s = q_s * 128.0 + c_s

        vrow = _t_to_row(v_s, eye)
        krow = _t_to_row(key_s, eye)
        valid = lanef < count2
        vrow = jnp.where(valid, vrow, NEGP)
        krow = jnp.where(valid, krow, 1e9)

        k = 2
        while k <= 128:
            j = k // 2
            while j >= 1:
                pv = jnp.where((lane & j) == 0,
                               pltpu.roll(vrow, 128 - j, 1),
                               pltpu.roll(vrow, j, 1))
                pk = jnp.where((lane & j) == 0,
                               pltpu.roll(krow, 128 - j, 1),
                               pltpu.roll(krow, j, 1))
                iamlow = (lane & j) == 0
                descb = (lane & k) == 0
                self_better = (vrow > pv) | ((vrow == pv) & (krow < pk))
                keep = iamlow == descb
                vrow = jnp.where(keep == self_better, vrow, pv)
                krow = jnp.where(keep == self_better, krow, pk)
                j //= 2
            k *= 2

        scores_ref[i] = vrow
        ki = krow.astype(jnp.int32)
        qi = ki // 128
        labels_ref[i] = ki - qi * 128

        q_t = _t_to_col(qi.astype(jnp.float32), eye)
        vg = jnp.zeros((128, V), jnp.float32)
        bg = jnp.zeros((128, 8), jnp.float32)
        for j in range(8):
            n = min(128, Q - j * 128)
            if n <= 0:
                break
            ln = lax.broadcasted_iota(jnp.int32, (1, n), 1).astype(jnp.float32)
            g = (q_t == ln + jnp.float32(128 * j)).astype(jnp.float32)
            vg = vg + DOT(g, verb_ref[i, j * 128:j * 128 + n, :],
                          (((1,), (0,)), ((), ())))
            bg = bg + DOT(g, boxes_ref[i, j * 128:j * 128 + n, :],
                          (((1,), (0,)), ((), ())))
        p_col = _t_to_col(vrow, eye)
        vs_ref[i] = p_col / (1.0 + jnp.exp(-vg))

        mi = lax.broadcasted_iota(jnp.int32, (8, 8), 0)
        mj = lax.broadcasted_iota(jnp.int32, (8, 8), 1)
        a = mj - (mi // 4) * 4
        sgn = jnp.where(mi % 4 < 2, -0.5, 0.5).astype(jnp.float32)
        M = (jnp.where(a == mi % 2, 1.0, 0.0)
             + jnp.where(a == mi % 2 + 2, sgn, 0.0)).astype(jnp.float32)
        conv = DOT(bg, M, (((1,), (1,)), ((), ())))
        boxes_o_ref[i] = conv * scale_ref[i]


@jax.jit
def kernel(pred_obj_logits, pred_verb_logits, pred_sub_boxes, pred_obj_boxes,
           target_sizes):
    B, Q, C = pred_obj_logits.shape
    V = pred_verb_logits.shape[-1]

    boxes8 = jnp.concatenate([pred_sub_boxes, pred_obj_boxes], axis=-1)
    h = target_sizes[:, 0].astype(jnp.float32)
    w = target_sizes[:, 1].astype(jnp.float32)
    scale8 = jnp.stack([w, h, w, h, w, h, w, h], axis=1)[:, None, :]

    scores_o, labels_o, vs_o, boxes_o = pl.pallas_call(
        _body,
        grid=(B // IMGS,),
        in_specs=[
            pl.BlockSpec((IMGS, Q, C), lambda b: (b, 0, 0)),
            pl.BlockSpec((IMGS, Q, V), lambda b: (b, 0, 0)),
            pl.BlockSpec((IMGS, Q, 8), lambda b: (b, 0, 0)),
            pl.BlockSpec((IMGS, 1, 8), lambda b: (b, 0, 0)),
        ],
        out_specs=[
            pl.BlockSpec((IMGS, 1, 128), lambda b: (b, 0, 0)),
            pl.BlockSpec((IMGS, 1, 128), lambda b: (b, 0, 0)),
            pl.BlockSpec((IMGS, 128, V), lambda b: (b, 0, 0)),
            pl.BlockSpec((IMGS, 128, 8), lambda b: (b, 0, 0)),
        ],
        out_shape=[
            jax.ShapeDtypeStruct((B, 1, 128), jnp.float32),
            jax.ShapeDtypeStruct((B, 1, 128), jnp.int32),
            jax.ShapeDtypeStruct((B, 128, V), jnp.float32),
            jax.ShapeDtypeStruct((B, 128, 8), jnp.float32),
        ],
        scratch_shapes=(
            [pltpu.VMEM((1024, 128), jnp.float32) for _ in range(IMGS)]
            + [pltpu.VMEM((1024, 1), jnp.float32) for _ in range(IMGS)]
        ),
        compiler_params=pltpu.CompilerParams(
            dimension_semantics=("parallel",)),
    )(pred_obj_logits, pred_verb_logits, boxes8, scale8)

    obj_scores = scores_o[:, 0, :K]
    obj_labels = labels_o[:, 0, :K]
    labels = jnp.concatenate(
        [jnp.full_like(obj_labels, SUBJ_ID), obj_labels], axis=1)
    bx = boxes_o[:, :K, :]
    boxes = jnp.concatenate([bx[:, :, 0:4], bx[:, :, 4:8]], axis=1)
    vs = vs_o[:, :K, :]
    ids = jnp.arange(2 * K)
    return labels, boxes, vs, obj_scores, ids[:K], ids[K:]
